# baseline (speedup 1.0000x reference)
import functools

import jax
import jax.numpy as jnp
from jax import lax
from jax.experimental import pallas as pl
from jax.experimental.pallas import tpu as pltpu
from jax.experimental.pallas import tpu_sc as plsc

K = 8192
D = 256
N = 65536
TN = 1024
PARTS = 2
NP = N // PARTS

NC = 2
NS = 16
NW = NC * NS
RCH = 128


CP = 2816
KP = 3 * CP


def _argmax_body(xt_ref, et_ref, xn_ref, en_ref, idx_ref):
    xt = xt_ref[...]
    m = jnp.dot(xt, et_ref[...], preferred_element_type=jnp.float32)
    u = (xn_ref[...].reshape(TN, 1) - 2.0 * m) + en_ref[...]

    def cmin(lo, hi):
        dm = u[:, lo:hi]
        return jnp.min(dm, axis=1), jnp.argmin(dm, axis=1)

    v0, i0 = cmin(0, CP)
    v1, i1 = cmin(CP, 2 * CP)
    v2, i2 = cmin(2 * CP, KP)
    accf = v0.astype(jnp.bfloat16).astype(jnp.float32)
    win1 = v1 < accf
    acc_i = jnp.where(win1, i1 + 2736, i0)
    accf = jnp.where(win1, v1, accf).astype(jnp.bfloat16).astype(jnp.float32)
    win2 = v2 < accf
    idx_ref[...] = jnp.where(win2, i2 + 5472, acc_i).astype(jnp.int32)


def _make_argmax(part):
    off = part * (NP // TN)
    return pl.pallas_call(
        _argmax_body,
        grid=(NP // TN,),
        in_specs=[pl.BlockSpec((TN, D), lambda i: (i + off, 0)),
                  pl.BlockSpec((D, KP), lambda i: (0, 0)),
                  pl.BlockSpec((TN,), lambda i: (i + off,)),
                  pl.BlockSpec((1, KP), lambda i: (0, 0))],
        out_specs=pl.BlockSpec((TN,), lambda i: (i,)),
        out_shape=jax.ShapeDtypeStruct((NP,), jnp.int32),
    )


_argmax_calls = [_make_argmax(p) for p in range(PARTS)]

BW = NP // NW
NCH = BW // RCH


def _sc_gather_hist(idx_hbm, embed_hbm, quant_hbm, hist_hbm,
                    idxv, buf, histv, sem):
    cid = lax.axis_index("c")
    sid = lax.axis_index("s")
    wid = sid * NC + cid

    pltpu.sync_copy(idx_hbm.at[pl.ds(wid * NCH, NCH)], idxv)

    def _zero(i, _):
        histv[pl.ds(i * 16, 16)] = jnp.zeros((16,), jnp.int32)
        return 0
    lax.fori_loop(0, K // 16, _zero, 0)

    ones = jnp.ones((16,), jnp.int32)

    def _chunk(c, _):
        pltpu.async_copy(embed_hbm.at[idxv.at[c]], buf, sem).wait()
        pltpu.sync_copy(buf, quant_hbm.at[pl.ds(wid * BW + c * RCH, RCH)])

        def _grp(g, _2):
            iv = idxv[c, pl.ds(g * 16, 16)]
            plsc.addupdate_scatter(histv, [iv], ones)
            return 0
        lax.fori_loop(0, RCH // 16, _grp, 0)
        return 0
    lax.fori_loop(0, NCH, _chunk, 0)

    pltpu.sync_copy(histv, hist_hbm.at[wid])


_sc_call = functools.partial(
    pl.kernel,
    mesh=plsc.VectorSubcoreMesh(core_axis_name="c", subcore_axis_name="s"),
    out_type=[jax.ShapeDtypeStruct((NP, D), jnp.float32),
              jax.ShapeDtypeStruct((NW, K), jnp.int32)],
    scratch_types=[pltpu.VMEM((NCH, RCH), jnp.int32),
                   pltpu.VMEM((RCH, D), jnp.float32),
                   pltpu.VMEM((K,), jnp.int32),
                   pltpu.SemaphoreType.DMA],
    compiler_params=pltpu.CompilerParams(needs_layout_passes=False),
)(_sc_gather_hist)


def _perp_body(h_ref, out_ref):
    counts = jnp.sum(h_ref[...].astype(jnp.float32), axis=0, keepdims=True)
    prob = counts * (1.0 / N)
    ent = jnp.sum(prob * jnp.log(prob + 1e-10), axis=1, keepdims=True)
    out_ref[...] = jnp.exp(-ent)


_perp_call = pl.pallas_call(
    _perp_body,
    in_specs=[pl.BlockSpec((PARTS * NW, K), lambda: (0, 0))],
    out_specs=pl.BlockSpec((1, 1), lambda: (0, 0)),
    out_shape=jax.ShapeDtypeStruct((1, 1), jnp.float32),
)


def kernel(x, embed):
    xf = x.reshape(N, D)
    et = embed.T
    xn = jnp.sum(x ** 2, axis=2).reshape(N)
    en = jnp.sum(et ** 2, axis=0).reshape(1, K)
    z = jnp.zeros((D, CP - 2736), jnp.float32)
    z2 = jnp.zeros((D, CP - 2720), jnp.float32)
    etp = jnp.concatenate([et[:, :2736], z, et[:, 2736:5472], z, et[:, 5472:], z2], axis=1)
    inf = jnp.full((1, CP - 2736), jnp.inf, jnp.float32)
    inf2 = jnp.full((1, CP - 2720), jnp.inf, jnp.float32)
    enp = jnp.concatenate([en[:, :2736], inf, en[:, 2736:5472], inf, en[:, 5472:], inf2], axis=1)
    quants, hists = [], []
    for p in range(PARTS):
        idx = _argmax_calls[p](xf, etp, xn, enp)
        q, h = _sc_call(idx.reshape(NP // RCH, RCH), embed)
        quants.append(q)
        hists.append(h)
    perp = _perp_call(jnp.concatenate(hists, axis=0))
    quant = jnp.concatenate(quants, axis=0)
    return quant.reshape(x.shape), perp.reshape(())


# speedup vs baseline: 1.0224x; 1.0224x over previous
import functools

import jax
import jax.numpy as jnp
from jax import lax
from jax.experimental import pallas as pl
from jax.experimental.pallas import tpu as pltpu
from jax.experimental.pallas import tpu_sc as plsc

K = 8192
D = 256
N = 65536
TN = 512
PARTS = 4
NP = N // PARTS

NC = 2
NS = 16
NW = NC * NS
RCH = 128


CP = 2816
KP = 3 * CP


def _argmax_body(xt_ref, et_ref, xn_ref, en_ref, idx_ref):
    xt = xt_ref[...]
    m = jnp.dot(xt, et_ref[...], preferred_element_type=jnp.float32)
    u = (xn_ref[...].reshape(TN, 1) - 2.0 * m) + en_ref[...]

    def cmin(lo, hi):
        dm = u[:, lo:hi]
        return jnp.min(dm, axis=1), jnp.argmin(dm, axis=1)

    v0, i0 = cmin(0, CP)
    v1, i1 = cmin(CP, 2 * CP)
    v2, i2 = cmin(2 * CP, KP)
    accf = v0.astype(jnp.bfloat16).astype(jnp.float32)
    win1 = v1 < accf
    acc_i = jnp.where(win1, i1 + 2736, i0)
    accf = jnp.where(win1, v1, accf).astype(jnp.bfloat16).astype(jnp.float32)
    win2 = v2 < accf
    idx_ref[...] = jnp.where(win2, i2 + 5472, acc_i).astype(jnp.int32)


def _make_argmax(part):
    off = part * (NP // TN)
    return pl.pallas_call(
        _argmax_body,
        grid=(NP // TN,),
        in_specs=[pl.BlockSpec((TN, D), lambda i: (i + off, 0)),
                  pl.BlockSpec((D, KP), lambda i: (0, 0)),
                  pl.BlockSpec((TN,), lambda i: (i + off,)),
                  pl.BlockSpec((1, KP), lambda i: (0, 0))],
        out_specs=pl.BlockSpec((TN,), lambda i: (i,)),
        out_shape=jax.ShapeDtypeStruct((NP,), jnp.int32),
    )


_argmax_calls = [_make_argmax(p) for p in range(PARTS)]

BW = NP // NW
NCH = BW // RCH


def _sc_gather_hist(idx_hbm, embed_hbm, quant_hbm, hist_hbm,
                    idxv, buf, histv, sem):
    cid = lax.axis_index("c")
    sid = lax.axis_index("s")
    wid = sid * NC + cid

    pltpu.sync_copy(idx_hbm.at[pl.ds(wid * NCH, NCH)], idxv)

    def _zero(i, _):
        histv[pl.ds(i * 16, 16)] = jnp.zeros((16,), jnp.int32)
        return 0
    lax.fori_loop(0, K // 16, _zero, 0)

    ones = jnp.ones((16,), jnp.int32)

    def _chunk(c, _):
        pltpu.async_copy(embed_hbm.at[idxv.at[c]], buf, sem).wait()
        pltpu.sync_copy(buf, quant_hbm.at[pl.ds(wid * BW + c * RCH, RCH)])

        def _grp(g, _2):
            iv = idxv[c, pl.ds(g * 16, 16)]
            plsc.addupdate_scatter(histv, [iv], ones)
            return 0
        lax.fori_loop(0, RCH // 16, _grp, 0)
        return 0
    lax.fori_loop(0, NCH, _chunk, 0)

    pltpu.sync_copy(histv, hist_hbm.at[wid])


_sc_call = functools.partial(
    pl.kernel,
    mesh=plsc.VectorSubcoreMesh(core_axis_name="c", subcore_axis_name="s"),
    out_type=[jax.ShapeDtypeStruct((NP, D), jnp.float32),
              jax.ShapeDtypeStruct((NW, K), jnp.int32)],
    scratch_types=[pltpu.VMEM((NCH, RCH), jnp.int32),
                   pltpu.VMEM((RCH, D), jnp.float32),
                   pltpu.VMEM((K,), jnp.int32),
                   pltpu.SemaphoreType.DMA],
    compiler_params=pltpu.CompilerParams(needs_layout_passes=False),
)(_sc_gather_hist)


def _perp_body(h_ref, out_ref):
    counts = jnp.sum(h_ref[...].astype(jnp.float32), axis=0, keepdims=True)
    prob = counts * (1.0 / N)
    ent = jnp.sum(prob * jnp.log(prob + 1e-10), axis=1, keepdims=True)
    out_ref[...] = jnp.exp(-ent)


_perp_call = pl.pallas_call(
    _perp_body,
    in_specs=[pl.BlockSpec((PARTS * NW, K), lambda: (0, 0))],
    out_specs=pl.BlockSpec((1, 1), lambda: (0, 0)),
    out_shape=jax.ShapeDtypeStruct((1, 1), jnp.float32),
)


def kernel(x, embed):
    xf = x.reshape(N, D)
    et = embed.T
    xn = jnp.sum(x ** 2, axis=2).reshape(N)
    en = jnp.sum(et ** 2, axis=0).reshape(1, K)
    z = jnp.zeros((D, CP - 2736), jnp.float32)
    z2 = jnp.zeros((D, CP - 2720), jnp.float32)
    etp = jnp.concatenate([et[:, :2736], z, et[:, 2736:5472], z, et[:, 5472:], z2], axis=1)
    inf = jnp.full((1, CP - 2736), jnp.inf, jnp.float32)
    inf2 = jnp.full((1, CP - 2720), jnp.inf, jnp.float32)
    enp = jnp.concatenate([en[:, :2736], inf, en[:, 2736:5472], inf, en[:, 5472:], inf2], axis=1)
    quants, hists = [], []
    for p in range(PARTS):
        idx = _argmax_calls[p](xf, etp, xn, enp)
        q, h = _sc_call(idx.reshape(NP // RCH, RCH), embed)
        quants.append(q)
        hists.append(h)
    perp = _perp_call(jnp.concatenate(hists, axis=0))
    quant = jnp.concatenate(quants, axis=0)
    return quant.reshape(x.shape), perp.reshape(())


# speedup vs baseline: 1.0245x; 1.0021x over previous
import functools

import jax
import jax.numpy as jnp
from jax import lax
from jax.experimental import pallas as pl
from jax.experimental.pallas import tpu as pltpu
from jax.experimental.pallas import tpu_sc as plsc

K = 8192
D = 256
N = 65536
TN = 512
PARTS = 8
NP = N // PARTS

NC = 2
NS = 16
NW = NC * NS
RCH = 128


CP = 2816
KP = 3 * CP


def _argmax_body(xt_ref, et_ref, xn_ref, en_ref, idx_ref):
    xt = xt_ref[...]
    m = jnp.dot(xt, et_ref[...], preferred_element_type=jnp.float32)
    u = (xn_ref[...].reshape(TN, 1) - 2.0 * m) + en_ref[...]

    def cmin(lo, hi):
        dm = u[:, lo:hi]
        return jnp.min(dm, axis=1), jnp.argmin(dm, axis=1)

    v0, i0 = cmin(0, CP)
    v1, i1 = cmin(CP, 2 * CP)
    v2, i2 = cmin(2 * CP, KP)
    accf = v0.astype(jnp.bfloat16).astype(jnp.float32)
    win1 = v1 < accf
    acc_i = jnp.where(win1, i1 + 2736, i0)
    accf = jnp.where(win1, v1, accf).astype(jnp.bfloat16).astype(jnp.float32)
    win2 = v2 < accf
    idx_ref[...] = jnp.where(win2, i2 + 5472, acc_i).astype(jnp.int32)


def _make_argmax(part):
    off = part * (NP // TN)
    return pl.pallas_call(
        _argmax_body,
        grid=(NP // TN,),
        in_specs=[pl.BlockSpec((TN, D), lambda i: (i + off, 0)),
                  pl.BlockSpec((D, KP), lambda i: (0, 0)),
                  pl.BlockSpec((TN,), lambda i: (i + off,)),
                  pl.BlockSpec((1, KP), lambda i: (0, 0))],
        out_specs=pl.BlockSpec((TN,), lambda i: (i,)),
        out_shape=jax.ShapeDtypeStruct((NP,), jnp.int32),
    )


_argmax_calls = [_make_argmax(p) for p in range(PARTS)]

BW = NP // NW
NCH = BW // RCH


def _sc_gather_hist(idx_hbm, embed_hbm, quant_hbm, hist_hbm,
                    idxv, buf, histv, sem):
    cid = lax.axis_index("c")
    sid = lax.axis_index("s")
    wid = sid * NC + cid

    pltpu.sync_copy(idx_hbm.at[pl.ds(wid * NCH, NCH)], idxv)

    def _zero(i, _):
        histv[pl.ds(i * 16, 16)] = jnp.zeros((16,), jnp.int32)
        return 0
    lax.fori_loop(0, K // 16, _zero, 0)

    ones = jnp.ones((16,), jnp.int32)

    def _chunk(c, _):
        pltpu.async_copy(embed_hbm.at[idxv.at[c]], buf, sem).wait()
        pltpu.sync_copy(buf, quant_hbm.at[pl.ds(wid * BW + c * RCH, RCH)])

        def _grp(g, _2):
            iv = idxv[c, pl.ds(g * 16, 16)]
            plsc.addupdate_scatter(histv, [iv], ones)
            return 0
        lax.fori_loop(0, RCH // 16, _grp, 0)
        return 0
    lax.fori_loop(0, NCH, _chunk, 0)

    pltpu.sync_copy(histv, hist_hbm.at[wid])


_sc_call = functools.partial(
    pl.kernel,
    mesh=plsc.VectorSubcoreMesh(core_axis_name="c", subcore_axis_name="s"),
    out_type=[jax.ShapeDtypeStruct((NP, D), jnp.float32),
              jax.ShapeDtypeStruct((NW, K), jnp.int32)],
    scratch_types=[pltpu.VMEM((NCH, RCH), jnp.int32),
                   pltpu.VMEM((RCH, D), jnp.float32),
                   pltpu.VMEM((K,), jnp.int32),
                   pltpu.SemaphoreType.DMA],
    compiler_params=pltpu.CompilerParams(needs_layout_passes=False),
)(_sc_gather_hist)


def _perp_body(h_ref, out_ref):
    counts = jnp.sum(h_ref[...].astype(jnp.float32), axis=0, keepdims=True)
    prob = counts * (1.0 / N)
    ent = jnp.sum(prob * jnp.log(prob + 1e-10), axis=1, keepdims=True)
    out_ref[...] = jnp.exp(-ent)


_perp_call = pl.pallas_call(
    _perp_body,
    in_specs=[pl.BlockSpec((PARTS * NW, K), lambda: (0, 0))],
    out_specs=pl.BlockSpec((1, 1), lambda: (0, 0)),
    out_shape=jax.ShapeDtypeStruct((1, 1), jnp.float32),
)


def kernel(x, embed):
    xf = x.reshape(N, D)
    et = embed.T
    xn = jnp.sum(x ** 2, axis=2).reshape(N)
    en = jnp.sum(et ** 2, axis=0).reshape(1, K)
    z = jnp.zeros((D, CP - 2736), jnp.float32)
    z2 = jnp.zeros((D, CP - 2720), jnp.float32)
    etp = jnp.concatenate([et[:, :2736], z, et[:, 2736:5472], z, et[:, 5472:], z2], axis=1)
    inf = jnp.full((1, CP - 2736), jnp.inf, jnp.float32)
    inf2 = jnp.full((1, CP - 2720), jnp.inf, jnp.float32)
    enp = jnp.concatenate([en[:, :2736], inf, en[:, 2736:5472], inf, en[:, 5472:], inf2], axis=1)
    quants, hists = [], []
    for p in range(PARTS):
        idx = _argmax_calls[p](xf, etp, xn, enp)
        q, h = _sc_call(idx.reshape(NP // RCH, RCH), embed)
        quants.append(q)
        hists.append(h)
    perp = _perp_call(jnp.concatenate(hists, axis=0))
    quant = jnp.concatenate(quants, axis=0)
    return quant.reshape(x.shape), perp.reshape(())


# speedup vs baseline: 1.0453x; 1.0203x over previous
import functools

import jax
import jax.numpy as jnp
from jax import lax
from jax.experimental import pallas as pl
from jax.experimental.pallas import tpu as pltpu
from jax.experimental.pallas import tpu_sc as plsc

K = 8192
D = 256
N = 65536
TN = 512
PARTS = 4
NP = N // PARTS

NC = 2
NS = 16
NW = NC * NS
RCH = 128


CP = 2816
KP = 3 * CP


def _argmax_body(xt_ref, et_ref, xn_ref, en_ref, idx_ref):
    xt = xt_ref[...]
    m2 = jnp.dot(xt, et_ref[...], preferred_element_type=jnp.float32)
    u = (xn_ref[...].reshape(TN, 1) - m2) + en_ref[...]
    ks = lax.broadcasted_iota(jnp.int32, (1, CP), 1)
    big = jnp.int32(2 ** 30)

    def cmin(lo, hi):
        dm = u[:, lo:hi]
        v = jnp.min(dm, axis=1)
        i = jnp.min(jnp.where(dm == v[:, None], ks, big), axis=1)
        return v, i

    v0, i0 = cmin(0, CP)
    v1, i1 = cmin(CP, 2 * CP)
    v2, i2 = cmin(2 * CP, KP)
    accf = v0.astype(jnp.bfloat16).astype(jnp.float32)
    win1 = v1 < accf
    acc_i = jnp.where(win1, i1 + 2736, i0)
    accf = jnp.where(win1, v1, accf).astype(jnp.bfloat16).astype(jnp.float32)
    win2 = v2 < accf
    idx_ref[...] = jnp.where(win2, i2 + 5472, acc_i).astype(jnp.int32)


def _make_argmax(part):
    off = part * (NP // TN)
    return pl.pallas_call(
        _argmax_body,
        grid=(NP // TN,),
        in_specs=[pl.BlockSpec((TN, D), lambda i: (i + off, 0)),
                  pl.BlockSpec((D, KP), lambda i: (0, 0)),
                  pl.BlockSpec((TN,), lambda i: (i + off,)),
                  pl.BlockSpec((1, KP), lambda i: (0, 0))],
        out_specs=pl.BlockSpec((TN,), lambda i: (i,)),
        out_shape=jax.ShapeDtypeStruct((NP,), jnp.int32),
    )


_argmax_calls = [_make_argmax(p) for p in range(PARTS)]

BW = NP // NW
NCH = BW // RCH


def _sc_gather_hist(idx_hbm, embed_hbm, quant_hbm, hist_hbm,
                    idxv, buf, histv, sem):
    cid = lax.axis_index("c")
    sid = lax.axis_index("s")
    wid = sid * NC + cid

    pltpu.sync_copy(idx_hbm.at[pl.ds(wid * NCH, NCH)], idxv)

    def _zero(i, _):
        histv[pl.ds(i * 16, 16)] = jnp.zeros((16,), jnp.int32)
        return 0
    lax.fori_loop(0, K // 16, _zero, 0)

    ones = jnp.ones((16,), jnp.int32)

    def _chunk(c, _):
        pltpu.async_copy(embed_hbm.at[idxv.at[c]], buf, sem).wait()
        pltpu.sync_copy(buf, quant_hbm.at[pl.ds(wid * BW + c * RCH, RCH)])

        def _grp(g, _2):
            iv = idxv[c, pl.ds(g * 16, 16)]
            plsc.addupdate_scatter(histv, [iv], ones)
            return 0
        lax.fori_loop(0, RCH // 16, _grp, 0)
        return 0
    lax.fori_loop(0, NCH, _chunk, 0)

    pltpu.sync_copy(histv, hist_hbm.at[wid])


_sc_call = functools.partial(
    pl.kernel,
    mesh=plsc.VectorSubcoreMesh(core_axis_name="c", subcore_axis_name="s"),
    out_type=[jax.ShapeDtypeStruct((NP, D), jnp.float32),
              jax.ShapeDtypeStruct((NW, K), jnp.int32)],
    scratch_types=[pltpu.VMEM((NCH, RCH), jnp.int32),
                   pltpu.VMEM((RCH, D), jnp.float32),
                   pltpu.VMEM((K,), jnp.int32),
                   pltpu.SemaphoreType.DMA],
    compiler_params=pltpu.CompilerParams(needs_layout_passes=False),
)(_sc_gather_hist)


def _perp_body(h_ref, out_ref):
    counts = jnp.sum(h_ref[...].astype(jnp.float32), axis=0, keepdims=True)
    prob = counts * (1.0 / N)
    ent = jnp.sum(prob * jnp.log(prob + 1e-10), axis=1, keepdims=True)
    out_ref[...] = jnp.exp(-ent)


_perp_call = pl.pallas_call(
    _perp_body,
    in_specs=[pl.BlockSpec((PARTS * NW, K), lambda: (0, 0))],
    out_specs=pl.BlockSpec((1, 1), lambda: (0, 0)),
    out_shape=jax.ShapeDtypeStruct((1, 1), jnp.float32),
)


def kernel(x, embed):
    xf = x.reshape(N, D)
    et = embed.T
    xn = jnp.sum(xf ** 2, axis=1, keepdims=True).reshape(N)
    en = jnp.sum(et ** 2, axis=0, keepdims=True)
    z = jnp.zeros((D, CP - 2736), jnp.float32)
    z2 = jnp.zeros((D, CP - 2720), jnp.float32)
    etp = jnp.concatenate([et[:, :2736], z, et[:, 2736:5472], z, et[:, 5472:], z2], axis=1)
    inf = jnp.full((1, CP - 2736), jnp.inf, jnp.float32)
    inf2 = jnp.full((1, CP - 2720), jnp.inf, jnp.float32)
    enp = jnp.concatenate([en[:, :2736], inf, en[:, 2736:5472], inf, en[:, 5472:], inf2], axis=1)
    etp = etp + etp
    quants, hists = [], []
    for p in range(PARTS):
        idx = _argmax_calls[p](xf, etp, xn, enp)
        q, h = _sc_call(idx.reshape(NP // RCH, RCH), embed)
        quants.append(q)
        hists.append(h)
    perp = _perp_call(jnp.concatenate(hists, axis=0))
    quant = jnp.concatenate(quants, axis=0)
    return quant.reshape(x.shape), perp.reshape(())
